# baseline (speedup 1.0000x reference)
import functools

import jax
import jax.numpy as jnp
from jax import lax
from jax.experimental import pallas as pl
from jax.experimental.pallas import tpu as pltpu
from jax.experimental.pallas import tpu_sc as plsc

NUM_ROWS = 1000000
BATCH = 16384
EMBED_DIM = 16
NPAIR = EMBED_DIM // 2
VEC = 16
ICHUNK = 128
CB = 131072
NB = -(-NUM_ROWS // CB)
TAIL = NUM_ROWS - (NB - 1) * CB
TAILP = -(-TAIL // 128) * 128
STRIDE = -(-NUM_ROWS // 128) * 128


def _detile_body(ut_ref, it_ref, ul_ref, il_ref, up_ref, ip_ref, sem):
    c = pl.program_id(0)
    base = c * CB

    def pack(src_ref, dst_ref):
        x16 = lax.bitcast_convert_type(
            src_ref[...].astype(jnp.bfloat16), jnp.uint16)
        x3 = x16.reshape(NPAIR, 2, CB)
        lo = x3[:, 0, :].astype(jnp.uint32)
        hi = x3[:, 1, :].astype(jnp.uint32)
        dst_ref[...] = lo | (hi << 16)

    pack(ut_ref, up_ref)
    pack(it_ref, ip_ref)

    def emit(width):
        copies = []
        for jp in range(NPAIR):
            copies.append(pltpu.async_copy(
                up_ref.at[jp, pl.ds(0, width)],
                ul_ref.at[pl.ds(jp * STRIDE + base, width)], sem))
            copies.append(pltpu.async_copy(
                ip_ref.at[jp, pl.ds(0, width)],
                il_ref.at[pl.ds(jp * STRIDE + base, width)], sem))
        for cp in copies:
            cp.wait()

    @pl.when(c < NB - 1)
    def _():
        emit(CB)

    @pl.when(c == NB - 1)
    def _():
        emit(TAILP)


@jax.jit
def _gmf(user, item, user_table, item_table):
    info = plsc.get_sparse_core_info()
    nc, ns = info.num_cores, info.num_subcores
    nw = nc * ns
    b_per_w = BATCH // nw
    n_chunks = b_per_w // ICHUNK

    utT = user_table.T
    itT = item_table.T

    ulin, ilin = pl.pallas_call(
        _detile_body,
        grid=(NB,),
        in_specs=[
            pl.BlockSpec((EMBED_DIM, CB), lambda c: (0, c)),
            pl.BlockSpec((EMBED_DIM, CB), lambda c: (0, c)),
        ],
        out_specs=[
            pl.BlockSpec(memory_space=pl.ANY),
            pl.BlockSpec(memory_space=pl.ANY),
        ],
        out_shape=[
            jax.ShapeDtypeStruct((STRIDE * NPAIR,), jnp.uint32),
            jax.ShapeDtypeStruct((STRIDE * NPAIR,), jnp.uint32),
        ],
        scratch_shapes=[
            pltpu.VMEM((NPAIR, CB), jnp.uint32),
            pltpu.VMEM((NPAIR, CB), jnp.uint32),
            pltpu.SemaphoreType.DMA,
        ],
        compiler_params=pltpu.CompilerParams(
            dimension_semantics=("arbitrary",)),
    )(utT, itT)

    mesh = plsc.VectorSubcoreMesh(core_axis_name="c", subcore_axis_name="s")

    @functools.partial(
        pl.kernel,
        out_type=jax.ShapeDtypeStruct((NPAIR, BATCH), jnp.uint32),
        mesh=mesh,
        compiler_params=pltpu.CompilerParams(needs_layout_passes=False),
        scratch_types=[
            pltpu.VMEM((n_chunks, ICHUNK), jnp.int32),
            pltpu.VMEM((n_chunks, ICHUNK), jnp.int32),
            pltpu.VMEM((NPAIR, n_chunks, ICHUNK), jnp.uint32),
            pltpu.VMEM((NPAIR, n_chunks, ICHUNK), jnp.uint32),
            pltpu.VMEM((NPAIR, b_per_w), jnp.uint32),
            pltpu.SemaphoreType.DMA,
            pltpu.SemaphoreType.DMA,
        ],
    )
    def gmf(user_hbm, item_hbm, ul_hbm, il_hbm, out_hbm,
            uidx_v, iidx_v, ug_v, ig_v, out_v, sem_u, sem_i):
        wid = lax.axis_index("s") * nc + lax.axis_index("c")
        base = wid * b_per_w
        for c in range(n_chunks):
            pltpu.sync_copy(
                user_hbm.at[pl.ds(base + c * ICHUNK, ICHUNK)], uidx_v.at[c])
            pltpu.sync_copy(
                item_hbm.at[pl.ds(base + c * ICHUNK, ICHUNK)], iidx_v.at[c])

        copies = []
        for jp in range(NPAIR):
            for c in range(n_chunks):
                copies.append(pltpu.async_copy(
                    ul_hbm.at[pl.ds(jp * STRIDE, NUM_ROWS)].at[uidx_v.at[c]],
                    ug_v.at[jp, c], sem_u))
                copies.append(pltpu.async_copy(
                    il_hbm.at[pl.ds(jp * STRIDE, NUM_ROWS)].at[iidx_v.at[c]],
                    ig_v.at[jp, c], sem_i))
        for cp in copies:
            cp.wait()

        for jp in range(NPAIR):
            def mul(v, _):
                c = lax.shift_right_logical(v, 3)
                o = lax.mul(lax.bitwise_and(v, 7), VEC)
                ub = plsc.bitcast(ug_v[jp, c, pl.ds(o, VEC)], jnp.bfloat16)
                ib = plsc.bitcast(ig_v[jp, c, pl.ds(o, VEC)], jnp.bfloat16)
                out_v[jp, pl.ds(lax.mul(v, VEC), VEC)] = plsc.bitcast(
                    ub * ib, jnp.uint32)
                return 0
            lax.fori_loop(0, b_per_w // VEC, mul, 0)

        pltpu.sync_copy(out_v, out_hbm.at[:, pl.ds(base, b_per_w)])

    out_pairs = gmf(user, item, ulin, ilin)
    out_bf = lax.bitcast_convert_type(out_pairs, jnp.bfloat16)
    return out_bf.transpose(1, 0, 2).reshape(BATCH, EMBED_DIM).astype(
        jnp.float32)


def kernel(user, item, user_table, item_table):
    return _gmf(user, item, user_table, item_table)


# speedup vs baseline: 1.1128x; 1.1128x over previous
import functools

import jax
import jax.numpy as jnp
from jax import lax
from jax.experimental import pallas as pl
from jax.experimental.pallas import tpu as pltpu
from jax.experimental.pallas import tpu_sc as plsc

NUM_ROWS = 1000000
BATCH = 16384
EMBED_DIM = 16
NPAIR = EMBED_DIM // 2
VEC = 16
ICHUNK = 128
CB = 131072
NB = -(-NUM_ROWS // CB)
TAIL = NUM_ROWS - (NB - 1) * CB
TAILP = -(-TAIL // 128) * 128
STRIDE = -(-NUM_ROWS // 128) * 128


def _detile_body(ut_ref, it_ref, ul_ref, il_ref, up_ref, ip_ref, sem):
    c = pl.program_id(0)
    base = c * CB

    def pack(src_ref, dst_ref):
        x16 = lax.bitcast_convert_type(
            src_ref[...].astype(jnp.bfloat16), jnp.uint16)
        for jp in range(NPAIR):
            lo = x16[2 * jp:2 * jp + 1, :].astype(jnp.uint32)
            hi = x16[2 * jp + 1:2 * jp + 2, :].astype(jnp.uint32)
            dst_ref[jp:jp + 1, :] = lo | (hi << 16)

    pack(ut_ref, up_ref)
    pack(it_ref, ip_ref)

    def emit(width):
        copies = []
        for jp in range(NPAIR):
            copies.append(pltpu.async_copy(
                up_ref.at[jp, pl.ds(0, width)],
                ul_ref.at[pl.ds(jp * STRIDE + base, width)], sem))
            copies.append(pltpu.async_copy(
                ip_ref.at[jp, pl.ds(0, width)],
                il_ref.at[pl.ds(jp * STRIDE + base, width)], sem))
        for cp in copies:
            cp.wait()

    @pl.when(c < NB - 1)
    def _():
        emit(CB)

    @pl.when(c == NB - 1)
    def _():
        emit(TAILP)


@jax.jit
def _gmf(user, item, user_table, item_table):
    info = plsc.get_sparse_core_info()
    nc, ns = info.num_cores, info.num_subcores
    nw = nc * ns
    b_per_w = BATCH // nw
    n_chunks = b_per_w // ICHUNK

    utT = user_table.T
    itT = item_table.T

    ulin, ilin = pl.pallas_call(
        _detile_body,
        grid=(NB,),
        in_specs=[
            pl.BlockSpec((EMBED_DIM, CB), lambda c: (0, c)),
            pl.BlockSpec((EMBED_DIM, CB), lambda c: (0, c)),
        ],
        out_specs=[
            pl.BlockSpec(memory_space=pl.ANY),
            pl.BlockSpec(memory_space=pl.ANY),
        ],
        out_shape=[
            jax.ShapeDtypeStruct((STRIDE * NPAIR,), jnp.uint32),
            jax.ShapeDtypeStruct((STRIDE * NPAIR,), jnp.uint32),
        ],
        scratch_shapes=[
            pltpu.VMEM((NPAIR, CB), jnp.uint32),
            pltpu.VMEM((NPAIR, CB), jnp.uint32),
            pltpu.SemaphoreType.DMA,
        ],
        compiler_params=pltpu.CompilerParams(
            dimension_semantics=("arbitrary",)),
    )(utT, itT)

    mesh = plsc.VectorSubcoreMesh(core_axis_name="c", subcore_axis_name="s")

    @functools.partial(
        pl.kernel,
        out_type=jax.ShapeDtypeStruct((NPAIR, BATCH), jnp.uint32),
        mesh=mesh,
        compiler_params=pltpu.CompilerParams(needs_layout_passes=False),
        scratch_types=[
            pltpu.VMEM((n_chunks, ICHUNK), jnp.int32),
            pltpu.VMEM((n_chunks, ICHUNK), jnp.int32),
            pltpu.VMEM((NPAIR, n_chunks, ICHUNK), jnp.uint32),
            pltpu.VMEM((NPAIR, n_chunks, ICHUNK), jnp.uint32),
            pltpu.VMEM((NPAIR, b_per_w), jnp.uint32),
            pltpu.SemaphoreType.DMA,
            pltpu.SemaphoreType.DMA,
        ],
    )
    def gmf(user_hbm, item_hbm, ul_hbm, il_hbm, out_hbm,
            uidx_v, iidx_v, ug_v, ig_v, out_v, sem_u, sem_i):
        wid = lax.axis_index("s") * nc + lax.axis_index("c")
        base = wid * b_per_w
        for c in range(n_chunks):
            pltpu.sync_copy(
                user_hbm.at[pl.ds(base + c * ICHUNK, ICHUNK)], uidx_v.at[c])
            pltpu.sync_copy(
                item_hbm.at[pl.ds(base + c * ICHUNK, ICHUNK)], iidx_v.at[c])

        copies = []
        for jp in range(NPAIR):
            for c in range(n_chunks):
                copies.append(pltpu.async_copy(
                    ul_hbm.at[pl.ds(jp * STRIDE, NUM_ROWS)].at[uidx_v.at[c]],
                    ug_v.at[jp, c], sem_u))
                copies.append(pltpu.async_copy(
                    il_hbm.at[pl.ds(jp * STRIDE, NUM_ROWS)].at[iidx_v.at[c]],
                    ig_v.at[jp, c], sem_i))
        for cp in copies:
            cp.wait()

        for jp in range(NPAIR):
            def mul(v, _):
                c = lax.shift_right_logical(v, 3)
                o = lax.mul(lax.bitwise_and(v, 7), VEC)
                ub = plsc.bitcast(ug_v[jp, c, pl.ds(o, VEC)], jnp.bfloat16)
                ib = plsc.bitcast(ig_v[jp, c, pl.ds(o, VEC)], jnp.bfloat16)
                out_v[jp, pl.ds(lax.mul(v, VEC), VEC)] = plsc.bitcast(
                    ub * ib, jnp.uint32)
                return 0
            lax.fori_loop(0, b_per_w // VEC, mul, 0)

        pltpu.sync_copy(out_v, out_hbm.at[:, pl.ds(base, b_per_w)])

    out_pairs = gmf(user, item, ulin, ilin)
    out_bf = lax.bitcast_convert_type(out_pairs, jnp.bfloat16)
    return out_bf.transpose(1, 0, 2).reshape(BATCH, EMBED_DIM).astype(
        jnp.float32)


def kernel(user, item, user_table, item_table):
    return _gmf(user, item, user_table, item_table)
